# baseline (speedup 1.0000x reference)
import jax
import jax.numpy as jnp
from jax.experimental import pallas as pl

_NEG_SLOPE = 0.1
_BLOCK_COLS = 4096


def _prelu_body(src_ref, snk_ref, out_src_ref, out_snk_ref):
    x = src_ref[...]
    out_src_ref[...] = jnp.where(x >= 0, x, _NEG_SLOPE * x)
    y = snk_ref[...]
    out_snk_ref[...] = jnp.where(y >= 0, y, _NEG_SLOPE * y)


def kernel(source_emb, sink_emb, source_edge_index, sink_edge_index,
           W_src, a_src_s, a_src_d, b_src,
           W_snk, a_snk_s, a_snk_d, b_snk):
    n, d = source_emb.shape
    src = source_emb.T
    snk = sink_emb.T
    grid = (pl.cdiv(n, _BLOCK_COLS),)
    spec = pl.BlockSpec((d, _BLOCK_COLS), lambda i: (0, i))
    out_src, out_snk = pl.pallas_call(
        _prelu_body,
        grid=grid,
        in_specs=[spec, spec],
        out_specs=[spec, spec],
        out_shape=[
            jax.ShapeDtypeStruct((d, n), source_emb.dtype),
            jax.ShapeDtypeStruct((d, n), sink_emb.dtype),
        ],
    )(src, snk)
    return (out_src.T, out_snk.T)


# speedup vs baseline: 1.4979x; 1.4979x over previous
import jax
import jax.numpy as jnp
from jax.experimental import pallas as pl

_NEG_SLOPE = 0.1
_BLOCK_COLS = 16384


def _prelu_body(src_ref, snk_ref, out_src_ref, out_snk_ref):
    x = src_ref[...]
    out_src_ref[...] = jnp.where(x >= 0, x, _NEG_SLOPE * x)
    y = snk_ref[...]
    out_snk_ref[...] = jnp.where(y >= 0, y, _NEG_SLOPE * y)


def kernel(source_emb, sink_emb, source_edge_index, sink_edge_index,
           W_src, a_src_s, a_src_d, b_src,
           W_snk, a_snk_s, a_snk_d, b_snk):
    n, d = source_emb.shape
    src = source_emb.T
    snk = sink_emb.T
    grid = (pl.cdiv(n, _BLOCK_COLS),)
    spec = pl.BlockSpec((d, _BLOCK_COLS), lambda i: (0, i))
    out_src, out_snk = pl.pallas_call(
        _prelu_body,
        grid=grid,
        in_specs=[spec, spec],
        out_specs=[spec, spec],
        out_shape=[
            jax.ShapeDtypeStruct((d, n), source_emb.dtype),
            jax.ShapeDtypeStruct((d, n), sink_emb.dtype),
        ],
    )(src, snk)
    return (out_src.T, out_snk.T)


# speedup vs baseline: 1.6611x; 1.1089x over previous
import jax
import jax.numpy as jnp
from jax.experimental import pallas as pl

_NEG_SLOPE = 0.1
_BLOCK_COLS = 32768


def _prelu_body(src_ref, snk_ref, out_src_ref, out_snk_ref):
    x = src_ref[...]
    out_src_ref[...] = jnp.where(x >= 0, x, _NEG_SLOPE * x)
    y = snk_ref[...]
    out_snk_ref[...] = jnp.where(y >= 0, y, _NEG_SLOPE * y)


def kernel(source_emb, sink_emb, source_edge_index, sink_edge_index,
           W_src, a_src_s, a_src_d, b_src,
           W_snk, a_snk_s, a_snk_d, b_snk):
    n, d = source_emb.shape
    src = source_emb.T
    snk = sink_emb.T
    grid = (pl.cdiv(n, _BLOCK_COLS),)
    spec = pl.BlockSpec((d, _BLOCK_COLS), lambda i: (0, i))
    out_src, out_snk = pl.pallas_call(
        _prelu_body,
        grid=grid,
        in_specs=[spec, spec],
        out_specs=[spec, spec],
        out_shape=[
            jax.ShapeDtypeStruct((d, n), source_emb.dtype),
            jax.ShapeDtypeStruct((d, n), sink_emb.dtype),
        ],
    )(src, snk)
    return (out_src.T, out_snk.T)
